# baseline (speedup 1.0000x reference)
import functools

import jax
import jax.numpy as jnp
from jax import lax
from jax.experimental import pallas as pl
from jax.experimental.pallas import tpu as pltpu
from jax.experimental.pallas import tpu_sc as plsc

_LANES = 16


def _make_sc_kernel(B, S, D, NC, NS):
    NW = NC * NS
    seq_per_tile = S // NW
    C = min(32, seq_per_tile)
    n_chunks = seq_per_tile // C
    vregs_per_row = D // _LANES

    mesh = plsc.VectorSubcoreMesh(core_axis_name="c", subcore_axis_name="s")

    @functools.partial(
        pl.kernel,
        mesh=mesh,
        out_type=jax.ShapeDtypeStruct((B * S, D), jnp.float32),
        scratch_types=[
            pltpu.VMEM((C,), jnp.int32),
            pltpu.VMEM((C, D), jnp.float32),
            pltpu.VMEM((C, D), jnp.float32),
            pltpu.SemaphoreType.DMA,
        ],
    )
    def sc_kernel(x_hbm, ids_hbm, pos_hbm, out_hbm, idx_v, pos_v, x_v, sem):
        wid = lax.axis_index("s") * NC + lax.axis_index("c")
        s0 = wid * seq_per_tile

        def chunk_body(i, _):
            sbase = s0 + i * C
            pltpu.sync_copy(ids_hbm.at[pl.ds(sbase, C)], idx_v)
            pltpu.async_copy(pos_hbm.at[idx_v], pos_v, sem).wait()

            def batch_body(b, _):
                row = b * S + sbase
                pltpu.async_copy(x_hbm.at[pl.ds(row, C)], x_v, sem).wait()

                def row_body(r, _):
                    for j in range(vregs_per_row):
                        v = pos_v[r, pl.ds(j * _LANES, _LANES)]
                        plsc.addupdate(x_v.at[r, pl.ds(j * _LANES, _LANES)], v)
                    return 0

                lax.fori_loop(0, C, row_body, 0)
                pltpu.sync_copy(x_v, out_hbm.at[pl.ds(row, C)])
                return 0

            lax.fori_loop(0, B, batch_body, 0)
            return 0

        lax.fori_loop(0, n_chunks, chunk_body, 0)

    return sc_kernel


def kernel(x, pos_table):
    B, S, D = x.shape
    info = plsc.get_sparse_core_info()
    sc = _make_sc_kernel(B, S, D, info.num_cores, info.num_subcores)
    positions = jnp.arange(S, dtype=jnp.int32)
    out = sc(x.reshape(B * S, D), positions, pos_table)
    return out.reshape(B, S, D)


# speedup vs baseline: 1.3036x; 1.3036x over previous
import functools

import jax
import jax.numpy as jnp
from jax import lax
from jax.experimental import pallas as pl
from jax.experimental.pallas import tpu as pltpu
from jax.experimental.pallas import tpu_sc as plsc

_LANES = 16


def _make_sc_kernel(B, S, D, NC, NS):
    NW = NC * NS
    seq_per_tile = S // NW
    C = min(32, seq_per_tile)
    n_chunks = seq_per_tile // C
    vregs_per_row = D // _LANES

    mesh = plsc.VectorSubcoreMesh(core_axis_name="c", subcore_axis_name="s")

    @functools.partial(
        pl.kernel,
        mesh=mesh,
        out_type=jax.ShapeDtypeStruct((B * S, D), jnp.float32),
        scratch_types=[
            pltpu.VMEM((C,), jnp.int32),
            pltpu.VMEM((C, D), jnp.float32),
            pltpu.VMEM((C, D), jnp.float32),
            pltpu.VMEM((C, D), jnp.float32),
            pltpu.SemaphoreType.DMA,
            pltpu.SemaphoreType.DMA,
            pltpu.SemaphoreType.DMA,
            pltpu.SemaphoreType.DMA,
            pltpu.SemaphoreType.DMA,
        ],
    )
    def sc_kernel(x_hbm, ids_hbm, pos_hbm, out_hbm,
                  idx_v, pos_v, x0, x1, sem_p, si0, si1, so0, so1):
        wid = lax.axis_index("s") * NC + lax.axis_index("c")
        s0 = wid * seq_per_tile
        xb = [x0, x1]
        sem_in = [si0, si1]
        sem_out = [so0, so1]
        steps = [(i, b) for i in range(n_chunks) for b in range(B)]

        def row_base(i, b):
            return b * S + s0 + i * C

        in_h = [None, None]
        out_h = [None, None]
        i0, b0 = steps[0]
        in_h[0] = pltpu.async_copy(
            x_hbm.at[pl.ds(row_base(i0, b0), C)], xb[0], sem_in[0])

        for k, (i, b) in enumerate(steps):
            cur = k % 2
            if b == 0:
                sbase = s0 + i * C
                pltpu.sync_copy(ids_hbm.at[pl.ds(sbase, C)], idx_v)
                pltpu.async_copy(pos_hbm.at[idx_v], pos_v, sem_p).wait()
            if k + 1 < len(steps):
                nxt = (k + 1) % 2
                ni, nb = steps[k + 1]
                if out_h[nxt] is not None:
                    out_h[nxt].wait()
                in_h[nxt] = pltpu.async_copy(
                    x_hbm.at[pl.ds(row_base(ni, nb), C)], xb[nxt], sem_in[nxt])
            in_h[cur].wait()

            def row_body(r, _, buf=xb[cur]):
                for j in range(vregs_per_row):
                    v = pos_v[r, pl.ds(j * _LANES, _LANES)]
                    plsc.addupdate(buf.at[r, pl.ds(j * _LANES, _LANES)], v)
                return 0

            lax.fori_loop(0, C, row_body, 0)
            out_h[cur] = pltpu.async_copy(
                xb[cur], out_hbm.at[pl.ds(row_base(i, b), C)], sem_out[cur])

        out_h[0].wait()
        out_h[1].wait()

    return sc_kernel


def kernel(x, pos_table):
    B, S, D = x.shape
    info = plsc.get_sparse_core_info()
    sc = _make_sc_kernel(B, S, D, info.num_cores, info.num_subcores)
    positions = jnp.arange(S, dtype=jnp.int32)
    out = sc(x.reshape(B * S, D), positions, pos_table)
    return out.reshape(B, S, D)
